# Initial kernel scaffold; baseline (speedup 1.0000x reference)
import jax
import jax.numpy as jnp
from jax.experimental import pallas as pl


def kernel(act_forward, emb_forward, act_back, emb_back, act_left, emb_left, act_right, emb_right, act_jump, emb_jump, act_sneak, emb_sneak, act_sprint, emb_sprint, act_attack, emb_attack, act_use, emb_use, act_drop, emb_drop, act_inventory, emb_inventory, act_hotbar_1, emb_hotbar_1, act_hotbar_2, emb_hotbar_2, act_hotbar_3, emb_hotbar_3, act_hotbar_4, emb_hotbar_4, act_hotbar_5, emb_hotbar_5, act_hotbar_6, emb_hotbar_6, act_hotbar_7, emb_hotbar_7, act_hotbar_8, emb_hotbar_8, act_hotbar_9, emb_hotbar_9, camera, W_cam, b_cam, W_final, b_final):
    raise NotImplementedError("write your pallas kernel here")


# speedup vs baseline: 6.0081x; 6.0081x over previous
import jax
import jax.numpy as jnp
from jax import lax
from jax.experimental import pallas as pl
from jax.experimental.pallas import tpu as pltpu
from jax.experimental.pallas import tpu_sc as plsc

_B = 16384
_D = 64
_N = 512
_NKEY = 20
_NCOL = 21
_CAM_COL = 11
_NBITS = 10
_T = 1 << _NBITS

_NC = 2
_NS = 16
_NW = _NC * _NS
_LPW = _B // _NW
_CH = 64
_NCHUNK = _LPW // _CH


def _fold_body(e0_ref, e1_ref, wc8_ref, bc_ref, a_ref, wf_ref, bf_ref,
               t0_ref, t1_ref, m_ref, idx_ref):
    c_acc = bf_ref[...]
    rows0 = []
    rows1 = []
    for c in range(_NCOL):
        blk = wf_ref[pl.ds(c * _D, _D), :]
        if c == _CAM_COL:
            m_ref[...] = jnp.dot(wc8_ref[...], blk,
                                 preferred_element_type=jnp.float32)
            c_acc = c_acc + jnp.dot(bc_ref[...], blk,
                                    preferred_element_type=jnp.float32)
        else:
            k = c if c < _CAM_COL else c - 1
            e0 = e0_ref[pl.ds(k, 1), :]
            de = e1_ref[pl.ds(k, 1), :] - e0
            c_acc = c_acc + jnp.dot(e0, blk, preferred_element_type=jnp.float32)
            row = jnp.dot(de, blk, preferred_element_type=jnp.float32)
            (rows0 if k < _NBITS else rows1).append(row)
    pad = jnp.zeros((16 - _NBITS, _N), jnp.float32)
    g0 = jnp.concatenate(rows0 + [pad], axis=0)
    g1 = jnp.concatenate(rows1 + [pad], axis=0)
    m_i = lax.broadcasted_iota(jnp.int32, (_T, 16), 0)
    j_i = lax.broadcasted_iota(jnp.int32, (_T, 16), 1)
    bits = ((m_i >> j_i) & 1).astype(jnp.float32)
    t0_ref[...] = jnp.dot(bits, g0, preferred_element_type=jnp.float32) + c_acc
    t1_ref[...] = jnp.dot(bits, g1, preferred_element_type=jnp.float32)
    r_i = lax.broadcasted_iota(jnp.int32, (8, _NKEY), 0)
    k_i = lax.broadcasted_iota(jnp.int32, (8, _NKEY), 1)
    sel = ((r_i == 0) & (k_i < _NBITS)) | ((r_i == 1) & (k_i >= _NBITS))
    sh = jnp.where(sel, jnp.where(r_i == 0, k_i, k_i - _NBITS), 0)
    pw = jnp.where(sel, jnp.left_shift(jnp.int32(1), sh), 0).astype(jnp.float32)
    idxf = jnp.dot(pw, a_ref[...], preferred_element_type=jnp.float32)
    idx_ref[...] = idxf.astype(jnp.int32)


_fold = pl.pallas_call(
    _fold_body,
    out_shape=(
        jax.ShapeDtypeStruct((_T, _N), jnp.float32),
        jax.ShapeDtypeStruct((_T, _N), jnp.float32),
        jax.ShapeDtypeStruct((8, _N), jnp.float32),
        jax.ShapeDtypeStruct((8, _B), jnp.int32),
    ),
)


def _sc_body(t0_hbm, t1_hbm, m_hbm, idx_hbm, cam_hbm, out_hbm,
             idx0_v, idx1_v, camx_v, camy_v, m_v, buf0, buf1, sem):
    cid = lax.axis_index("c")
    sid = lax.axis_index("s")
    base = (sid * _NC + cid) * _LPW
    pltpu.sync_copy(m_hbm.at[pl.ds(0, 2)], m_v)
    lanes = lax.iota(jnp.int32, 16)
    for g in range(_NCHUNK):
        gb = base + g * _CH
        pltpu.sync_copy(idx_hbm.at[0, pl.ds(gb, _CH)], idx0_v)
        pltpu.sync_copy(idx_hbm.at[1, pl.ds(gb, _CH)], idx1_v)
        pltpu.sync_copy(cam_hbm.at[0, pl.ds(gb, _CH)], camx_v)
        pltpu.sync_copy(cam_hbm.at[1, pl.ds(gb, _CH)], camy_v)
        pltpu.async_copy(t0_hbm.at[idx0_v], buf0, sem).wait()
        pltpu.async_copy(t1_hbm.at[idx1_v], buf1, sem).wait()

        def row(r, carry):
            r16 = jnp.full((16,), r, jnp.int32)
            cx = plsc.load_gather(camx_v, [r16])
            cy = plsc.load_gather(camy_v, [r16])
            for cc in range(_N // 16):
                cols = cc * 16 + lanes
                m0 = m_v[0, pl.ds(cc * 16, 16)]
                m1 = m_v[1, pl.ds(cc * 16, 16)]
                v0 = plsc.load_gather(buf0, [r16, cols])
                v1 = plsc.load_gather(buf1, [r16, cols])
                plsc.store_scatter(buf0, [r16, cols],
                                   v0 + v1 + cx * m0 + cy * m1)
            return carry

        lax.fori_loop(0, _CH, row, 0)
        pltpu.sync_copy(buf0, out_hbm.at[pl.ds(gb, _CH)])


_lookup = pl.kernel(
    _sc_body,
    out_type=jax.ShapeDtypeStruct((_B, _N), jnp.float32),
    mesh=plsc.VectorSubcoreMesh(core_axis_name="c", subcore_axis_name="s"),
    compiler_params=pltpu.CompilerParams(needs_layout_passes=False),
    scratch_types=[
        pltpu.VMEM((_CH,), jnp.int32),
        pltpu.VMEM((_CH,), jnp.int32),
        pltpu.VMEM((_CH,), jnp.float32),
        pltpu.VMEM((_CH,), jnp.float32),
        pltpu.VMEM((2, _N), jnp.float32),
        pltpu.VMEM((_CH, _N), jnp.float32),
        pltpu.VMEM((_CH, _N), jnp.float32),
        pltpu.SemaphoreType.DMA,
    ],
)


def kernel(act_forward, emb_forward, act_back, emb_back, act_left, emb_left,
           act_right, emb_right, act_jump, emb_jump, act_sneak, emb_sneak,
           act_sprint, emb_sprint, act_attack, emb_attack, act_use, emb_use,
           act_drop, emb_drop, act_inventory, emb_inventory,
           act_hotbar_1, emb_hotbar_1, act_hotbar_2, emb_hotbar_2,
           act_hotbar_3, emb_hotbar_3, act_hotbar_4, emb_hotbar_4,
           act_hotbar_5, emb_hotbar_5, act_hotbar_6, emb_hotbar_6,
           act_hotbar_7, emb_hotbar_7, act_hotbar_8, emb_hotbar_8,
           act_hotbar_9, emb_hotbar_9, camera, W_cam, b_cam, W_final, b_final):
    acts = [act_forward, act_back, act_left, act_right, act_jump, act_sneak,
            act_sprint, act_attack, act_use, act_drop, act_inventory,
            act_hotbar_1, act_hotbar_2, act_hotbar_3, act_hotbar_4,
            act_hotbar_5, act_hotbar_6, act_hotbar_7, act_hotbar_8,
            act_hotbar_9]
    embs = [emb_forward, emb_back, emb_left, emb_right, emb_jump, emb_sneak,
            emb_sprint, emb_attack, emb_use, emb_drop, emb_inventory,
            emb_hotbar_1, emb_hotbar_2, emb_hotbar_3, emb_hotbar_4,
            emb_hotbar_5, emb_hotbar_6, emb_hotbar_7, emb_hotbar_8,
            emb_hotbar_9]
    a = jnp.stack(acts, axis=0).astype(jnp.float32)
    e = jnp.stack(embs, axis=0)
    e0 = e[:, 0, :]
    e1 = e[:, 1, :]
    wc8 = jnp.zeros((8, _D), jnp.float32).at[0:2, :].set(W_cam)
    bc = b_cam.reshape(1, _D)
    bf = b_final.reshape(1, _N)
    t0, t1, m8, idx = _fold(e0, e1, wc8, bc, a, W_final, bf)
    return _lookup(t0, t1, m8, idx, jnp.transpose(camera))


# speedup vs baseline: 6.8922x; 1.1471x over previous
import jax
import jax.numpy as jnp
from jax import lax
from jax.experimental import pallas as pl
from jax.experimental.pallas import tpu as pltpu
from jax.experimental.pallas import tpu_sc as plsc

_B = 16384
_D = 64
_N = 512
_NKEY = 20
_NCOL = 21
_CAM_COL = 11
_NBITS = 10
_T = 1 << _NBITS

_NC = 2
_NS = 16
_NW = _NC * _NS
_LPW = _B // _NW
_CH = 64
_NCHUNK = _LPW // _CH


def _fold_body(e0_ref, e1_ref, wc8_ref, bc_ref, a_ref, wf_ref, bf_ref,
               t0_ref, t1_ref, m_ref, idx_ref):
    c_acc = bf_ref[...]
    rows0 = []
    rows1 = []
    for c in range(_NCOL):
        blk = wf_ref[pl.ds(c * _D, _D), :]
        if c == _CAM_COL:
            m_ref[...] = jnp.dot(wc8_ref[...], blk,
                                 preferred_element_type=jnp.float32)
            c_acc = c_acc + jnp.dot(bc_ref[...], blk,
                                    preferred_element_type=jnp.float32)
        else:
            k = c if c < _CAM_COL else c - 1
            e0 = e0_ref[pl.ds(k, 1), :]
            de = e1_ref[pl.ds(k, 1), :] - e0
            c_acc = c_acc + jnp.dot(e0, blk, preferred_element_type=jnp.float32)
            row = jnp.dot(de, blk, preferred_element_type=jnp.float32)
            (rows0 if k < _NBITS else rows1).append(row)
    pad = jnp.zeros((16 - _NBITS, _N), jnp.float32)
    g0 = jnp.concatenate(rows0 + [pad], axis=0)
    g1 = jnp.concatenate(rows1 + [pad], axis=0)
    m_i = lax.broadcasted_iota(jnp.int32, (_T, 16), 0)
    j_i = lax.broadcasted_iota(jnp.int32, (_T, 16), 1)
    bits = ((m_i >> j_i) & 1).astype(jnp.float32)
    t0_ref[...] = jnp.dot(bits, g0, preferred_element_type=jnp.float32) + c_acc
    t1_ref[...] = jnp.dot(bits, g1, preferred_element_type=jnp.float32)
    r_i = lax.broadcasted_iota(jnp.int32, (8, _NKEY), 0)
    k_i = lax.broadcasted_iota(jnp.int32, (8, _NKEY), 1)
    sel = ((r_i == 0) & (k_i < _NBITS)) | ((r_i == 1) & (k_i >= _NBITS))
    sh = jnp.where(sel, jnp.where(r_i == 0, k_i, k_i - _NBITS), 0)
    pw = jnp.where(sel, jnp.left_shift(jnp.int32(1), sh), 0).astype(jnp.float32)
    idxf = jnp.dot(pw, a_ref[...], preferred_element_type=jnp.float32)
    idx_ref[...] = idxf.astype(jnp.int32)


_fold = pl.pallas_call(
    _fold_body,
    out_shape=(
        jax.ShapeDtypeStruct((_T, _N), jnp.float32),
        jax.ShapeDtypeStruct((_T, _N), jnp.float32),
        jax.ShapeDtypeStruct((8, _N), jnp.float32),
        jax.ShapeDtypeStruct((8, _B), jnp.int32),
    ),
)


def _sc_body(t0_hbm, t1_hbm, m_hbm, idx_hbm, cam_hbm, out_hbm,
             idx0_all, idx1_all, camx_all, camy_all, m_v,
             buf0a, buf1a, buf0b, gsem_a, gsem_b, osem_a, osem_b):
    cid = lax.axis_index("c")
    sid = lax.axis_index("s")
    base = (sid * _NC + cid) * _LPW
    pltpu.sync_copy(m_hbm.at[pl.ds(0, 2)], m_v)
    pltpu.sync_copy(idx_hbm.at[0, pl.ds(base, _LPW)], idx0_all)
    pltpu.sync_copy(idx_hbm.at[1, pl.ds(base, _LPW)], idx1_all)
    pltpu.sync_copy(cam_hbm.at[0, pl.ds(base, _LPW)], camx_all)
    pltpu.sync_copy(cam_hbm.at[1, pl.ds(base, _LPW)], camy_all)
    lanes = lax.iota(jnp.int32, 16)
    ring = [buf0a, buf1a, buf0b]
    gsems = [gsem_a, gsem_b, osem_b]

    def ab(g):
        return ring[(2 * g) % 3], ring[(2 * g + 1) % 3]

    def issue_t0(g):
        return pltpu.async_copy(t0_hbm.at[idx0_all.at[pl.ds(g * _CH, _CH)]],
                                ab(g)[0], gsems[(2 * g) % 3])

    def issue_t1(g):
        return pltpu.async_copy(t1_hbm.at[idx1_all.at[pl.ds(g * _CH, _CH)]],
                                ab(g)[1], gsems[(2 * g + 1) % 3])

    pend = {g: {} for g in range(_NCHUNK)}
    pend[0]["t0"] = issue_t0(0)
    pend[0]["t1"] = issue_t1(0)
    out_pend = {}
    for g in range(_NCHUNK):
        b0, b1 = ab(g)
        pend[g].pop("t0").wait()
        pend[g].pop("t1").wait()

        def compute_half(half):
            hbase = half * (_N // 2)
            m0 = [m_v[0, pl.ds(hbase + cc * 16, 16)] for cc in range(16)]
            m1 = [m_v[1, pl.ds(hbase + cc * 16, 16)] for cc in range(16)]

            def row(r, carry):
                gr16 = jnp.full((16,), g * _CH + r, jnp.int32)
                r16 = jnp.full((16,), r, jnp.int32)
                cx = plsc.load_gather(camx_all, [gr16])
                cy = plsc.load_gather(camy_all, [gr16])
                for cc in range(16):
                    cols = (hbase + cc * 16) + lanes
                    v0 = plsc.load_gather(b0, [r16, cols])
                    v1 = plsc.load_gather(b1, [r16, cols])
                    plsc.store_scatter(b1, [r16, cols],
                                       v0 + v1 + cx * m0[cc] + cy * m1[cc])
                return carry

            lax.fori_loop(0, _CH, row, 0)

        compute_half(0)
        if g + 1 < _NCHUNK:
            if g - 1 in out_pend:
                out_pend.pop(g - 1).wait()
            pend[g + 1]["t0"] = issue_t0(g + 1)
        compute_half(1)
        if g + 1 < _NCHUNK:
            pend[g + 1]["t1"] = issue_t1(g + 1)
        out_pend[g] = pltpu.async_copy(
            b1, out_hbm.at[pl.ds(base + g * _CH, _CH)], osem_a)
    for g in sorted(out_pend):
        out_pend[g].wait()


_lookup = pl.kernel(
    _sc_body,
    out_type=jax.ShapeDtypeStruct((_B, _N), jnp.float32),
    mesh=plsc.VectorSubcoreMesh(core_axis_name="c", subcore_axis_name="s"),
    compiler_params=pltpu.CompilerParams(needs_layout_passes=False),
    scratch_types=[
        pltpu.VMEM((_LPW,), jnp.int32),
        pltpu.VMEM((_LPW,), jnp.int32),
        pltpu.VMEM((_LPW,), jnp.float32),
        pltpu.VMEM((_LPW,), jnp.float32),
        pltpu.VMEM((2, _N), jnp.float32),
        pltpu.VMEM((_CH, _N), jnp.float32),
        pltpu.VMEM((_CH, _N), jnp.float32),
        pltpu.VMEM((_CH, _N), jnp.float32),
        pltpu.SemaphoreType.DMA,
        pltpu.SemaphoreType.DMA,
        pltpu.SemaphoreType.DMA,
        pltpu.SemaphoreType.DMA,
    ],
)


def kernel(act_forward, emb_forward, act_back, emb_back, act_left, emb_left,
           act_right, emb_right, act_jump, emb_jump, act_sneak, emb_sneak,
           act_sprint, emb_sprint, act_attack, emb_attack, act_use, emb_use,
           act_drop, emb_drop, act_inventory, emb_inventory,
           act_hotbar_1, emb_hotbar_1, act_hotbar_2, emb_hotbar_2,
           act_hotbar_3, emb_hotbar_3, act_hotbar_4, emb_hotbar_4,
           act_hotbar_5, emb_hotbar_5, act_hotbar_6, emb_hotbar_6,
           act_hotbar_7, emb_hotbar_7, act_hotbar_8, emb_hotbar_8,
           act_hotbar_9, emb_hotbar_9, camera, W_cam, b_cam, W_final, b_final):
    acts = [act_forward, act_back, act_left, act_right, act_jump, act_sneak,
            act_sprint, act_attack, act_use, act_drop, act_inventory,
            act_hotbar_1, act_hotbar_2, act_hotbar_3, act_hotbar_4,
            act_hotbar_5, act_hotbar_6, act_hotbar_7, act_hotbar_8,
            act_hotbar_9]
    embs = [emb_forward, emb_back, emb_left, emb_right, emb_jump, emb_sneak,
            emb_sprint, emb_attack, emb_use, emb_drop, emb_inventory,
            emb_hotbar_1, emb_hotbar_2, emb_hotbar_3, emb_hotbar_4,
            emb_hotbar_5, emb_hotbar_6, emb_hotbar_7, emb_hotbar_8,
            emb_hotbar_9]
    a = jnp.stack(acts, axis=0).astype(jnp.float32)
    e = jnp.stack(embs, axis=0)
    e0 = e[:, 0, :]
    e1 = e[:, 1, :]
    wc8 = jnp.zeros((8, _D), jnp.float32).at[0:2, :].set(W_cam)
    bc = b_cam.reshape(1, _D)
    bf = b_final.reshape(1, _N)
    t0, t1, m8, idx = _fold(e0, e1, wc8, bc, a, W_final, bf)
    return _lookup(t0, t1, m8, idx, jnp.transpose(camera))
